# baseline (speedup 1.0000x reference)
import functools

import jax
import jax.numpy as jnp
import numpy as np
from jax import lax
from jax.experimental import pallas as pl
from jax.experimental.pallas import tpu as pltpu
from jax.experimental.pallas import tpu_sc as plsc

_ALPHA = 4
_NC, _NS = 2, 16
_NW = _NC * _NS


def _tc_fast_copy(frames):
    C, T, H, W = frames.shape
    n = T // _ALPHA

    def _body(in_hbm, fast_hbm, bufs, sem_in, sem_fast):
        def in_dma(j):
            return pltpu.make_async_copy(
                in_hbm.at[:, pl.ds(j * _ALPHA, _ALPHA)],
                bufs.at[j],
                sem_in.at[j],
            )

        def fast_dma(j):
            return pltpu.make_async_copy(
                bufs.at[j],
                fast_hbm.at[:, pl.ds(j * _ALPHA, _ALPHA)],
                sem_fast.at[j],
            )

        for j in range(n):
            in_dma(j).start()
        for j in range(n):
            in_dma(j).wait()
            fast_dma(j).start()
        for j in range(n):
            fast_dma(j).wait()

    return pl.pallas_call(
        _body,
        in_specs=[pl.BlockSpec(memory_space=pltpu.MemorySpace.HBM)],
        out_specs=pl.BlockSpec(memory_space=pltpu.MemorySpace.HBM),
        out_shape=jax.ShapeDtypeStruct((C, T, H, W), frames.dtype),
        scratch_shapes=[
            pltpu.VMEM((n, C, _ALPHA, H, W), frames.dtype),
            pltpu.SemaphoreType.DMA((n,)),
            pltpu.SemaphoreType.DMA((n,)),
        ],
    )(frames)


def _sc_gather(frames, C, T, n):
    _, _, H, W = frames.shape
    planes = C * n
    nchunk = 1
    while (planes * nchunk) % _NW or H % nchunk or (H // nchunk) % 8:
        nchunk += 1
    hh = H // nchunk
    tpw = planes * nchunk // _NW

    mesh = plsc.VectorSubcoreMesh(core_axis_name="c", subcore_axis_name="s")

    @functools.partial(
        pl.kernel,
        mesh=mesh,
        out_type=jax.ShapeDtypeStruct((C, n, H, W), frames.dtype),
        scratch_types=[
            pltpu.VMEM((tpw, hh, W), frames.dtype),
            pltpu.SemaphoreType.DMA((tpw,)),
            pltpu.SemaphoreType.DMA((tpw,)),
        ],
        compiler_params=pltpu.CompilerParams(use_tc_tiling_on_sc=True),
    )
    def k(in_hbm, out_hbm, bufs, sem_in, sem_out):
        wid = lax.axis_index("s") * _NC + lax.axis_index("c")

        def task(i):
            t = wid * tpw + i
            plane = t // nchunk
            chunk = t % nchunk
            j = plane % n
            c = plane // n
            src = (j * (T - 1)) // (n - 1)
            return c, src, j, chunk * hh

        def in_dma(i):
            c, src, _, h0 = task(i)
            return pltpu.make_async_copy(
                in_hbm.at[c, src, pl.ds(h0, hh)], bufs.at[i], sem_in.at[i]
            )

        def out_dma(i):
            c, _, j, h0 = task(i)
            return pltpu.make_async_copy(
                bufs.at[i], out_hbm.at[c, j, pl.ds(h0, hh)], sem_out.at[i]
            )

        for i in range(tpw):
            in_dma(i).start()
        for i in range(tpw):
            in_dma(i).wait()
            out_dma(i).start()
        for i in range(tpw):
            out_dma(i).wait()

    return k(frames)


def kernel(frames):
    C, T, H, W = frames.shape
    n = T // _ALPHA
    idx = np.linspace(0.0, T - 1, n).astype(np.int32)
    assert all(int(t) == (j * (T - 1)) // (n - 1) for j, t in enumerate(idx))

    slow = _sc_gather(frames, C, T, n)
    fast = _tc_fast_copy(frames)
    return (slow, fast)


# speedup vs baseline: 1.0055x; 1.0055x over previous
import functools

import jax
import jax.numpy as jnp
import numpy as np
from jax import lax
from jax.experimental import pallas as pl
from jax.experimental.pallas import tpu as pltpu
from jax.experimental.pallas import tpu_sc as plsc

_ALPHA = 4
_NC, _NS = 2, 16
_NW = _NC * _NS


def _tc_fast_copy(frames):
    C, T, H, W = frames.shape
    n = T // _ALPHA

    def _body(in_hbm, fast_hbm, bufs, sem_in, sem_fast):
        def in_dma(j):
            return pltpu.make_async_copy(
                in_hbm.at[:, pl.ds(j * _ALPHA, _ALPHA)],
                bufs.at[j],
                sem_in.at[j],
            )

        def fast_dma(j):
            return pltpu.make_async_copy(
                bufs.at[j],
                fast_hbm.at[:, pl.ds(j * _ALPHA, _ALPHA)],
                sem_fast.at[j],
            )

        for j in range(n):
            in_dma(j).start()
        for j in range(n):
            in_dma(j).wait()
            fast_dma(j).start()
        for j in range(n):
            fast_dma(j).wait()

    return pl.pallas_call(
        _body,
        in_specs=[pl.BlockSpec(memory_space=pltpu.MemorySpace.HBM)],
        out_specs=pl.BlockSpec(memory_space=pltpu.MemorySpace.HBM),
        out_shape=jax.ShapeDtypeStruct((C, T, H, W), frames.dtype),
        scratch_shapes=[
            pltpu.VMEM((n, C, _ALPHA, H, W), frames.dtype),
            pltpu.SemaphoreType.DMA((n,)),
            pltpu.SemaphoreType.DMA((n,)),
        ],
    )(frames)


def _sc_gather(frames, C, T, n):
    _, _, H, W = frames.shape
    planes = C * n
    nchunk = 1
    hh = H // nchunk
    tpw = -(-planes * nchunk // _NW)

    mesh = plsc.VectorSubcoreMesh(core_axis_name="c", subcore_axis_name="s")

    @functools.partial(
        pl.kernel,
        mesh=mesh,
        out_type=jax.ShapeDtypeStruct((C, n, H, W), frames.dtype),
        scratch_types=[
            pltpu.VMEM((tpw, hh, W), frames.dtype),
            pltpu.SemaphoreType.DMA((tpw,)),
            pltpu.SemaphoreType.DMA((tpw,)),
        ],
        compiler_params=pltpu.CompilerParams(use_tc_tiling_on_sc=True),
    )
    def k(in_hbm, out_hbm, bufs, sem_in, sem_out):
        wid = lax.axis_index("s") * _NC + lax.axis_index("c")

        def task(i):
            t = wid * tpw + i
            plane = t // nchunk
            chunk = t % nchunk
            j = plane % n
            c = plane // n
            src = (j * (T - 1)) // (n - 1)
            return c, src, j, chunk * hh

        def in_dma(i):
            c, src, _, h0 = task(i)
            return pltpu.make_async_copy(
                in_hbm.at[c, src, pl.ds(h0, hh)], bufs.at[i], sem_in.at[i]
            )

        def out_dma(i):
            c, _, j, h0 = task(i)
            return pltpu.make_async_copy(
                bufs.at[i], out_hbm.at[c, j, pl.ds(h0, hh)], sem_out.at[i]
            )

        @pl.when(wid * tpw < planes * nchunk)
        def _():
            for i in range(tpw):
                in_dma(i).start()
            for i in range(tpw):
                in_dma(i).wait()
                out_dma(i).start()
            for i in range(tpw):
                out_dma(i).wait()

    return k(frames)


def kernel(frames):
    C, T, H, W = frames.shape
    n = T // _ALPHA
    idx = np.linspace(0.0, T - 1, n).astype(np.int32)
    assert all(int(t) == (j * (T - 1)) // (n - 1) for j, t in enumerate(idx))

    slow = _sc_gather(frames, C, T, n)
    fast = _tc_fast_copy(frames)
    return (slow, fast)


# speedup vs baseline: 1.0116x; 1.0060x over previous
import functools

import jax
import jax.numpy as jnp
import numpy as np
from jax import lax
from jax.experimental import pallas as pl
from jax.experimental.pallas import tpu as pltpu
from jax.experimental.pallas import tpu_sc as plsc

_ALPHA = 4
_NC, _NS = 2, 16
_NW = _NC * _NS


def _tc_fast_copy(frames):
    C, T, H, W = frames.shape
    n = T // _ALPHA

    def _body(in_hbm, fast_hbm, bufs, sem_in, sem_fast):
        def in_dma(j):
            return pltpu.make_async_copy(
                in_hbm.at[:, pl.ds(j * _ALPHA, _ALPHA)],
                bufs.at[j],
                sem_in.at[j],
            )

        def fast_dma(j):
            return pltpu.make_async_copy(
                bufs.at[j],
                fast_hbm.at[:, pl.ds(j * _ALPHA, _ALPHA)],
                sem_fast.at[j],
            )

        for j in range(n):
            in_dma(j).start()
        for j in range(n):
            in_dma(j).wait()
            fast_dma(j).start()
        for j in range(n):
            fast_dma(j).wait()

    return pl.pallas_call(
        _body,
        in_specs=[pl.BlockSpec(memory_space=pltpu.MemorySpace.HBM)],
        out_specs=pl.BlockSpec(memory_space=pltpu.MemorySpace.HBM),
        out_shape=jax.ShapeDtypeStruct((C, T, H, W), frames.dtype),
        scratch_shapes=[
            pltpu.VMEM((n, C, _ALPHA, H, W), frames.dtype),
            pltpu.SemaphoreType.DMA((n,)),
            pltpu.SemaphoreType.DMA((n,)),
        ],
    )(frames)


def _sc_gather(frames, C, T, n):
    _, _, H, W = frames.shape
    planes = C * n
    nchunk = 1
    while (planes * nchunk) % _NW or H % nchunk or (H // nchunk) % 8:
        nchunk += 1
    hh = H // nchunk
    tpw = planes * nchunk // _NW

    mesh = plsc.VectorSubcoreMesh(core_axis_name="c", subcore_axis_name="s")

    @functools.partial(
        pl.kernel,
        mesh=mesh,
        out_type=jax.ShapeDtypeStruct((C, n, H, W), frames.dtype),
        scratch_types=[
            pltpu.VMEM_SHARED((_NS, tpw, hh, W), frames.dtype),
            pltpu.SemaphoreType.DMA((tpw,)),
            pltpu.SemaphoreType.DMA((tpw,)),
        ],
        compiler_params=pltpu.CompilerParams(use_tc_tiling_on_sc=True),
    )
    def k(in_hbm, out_hbm, bufs, sem_in, sem_out):
        sid = lax.axis_index("s")
        wid = sid * _NC + lax.axis_index("c")

        def task(i):
            t = wid * tpw + i
            plane = t // nchunk
            chunk = t % nchunk
            j = plane % n
            c = plane // n
            src = (j * (T - 1)) // (n - 1)
            return c, src, j, chunk * hh

        def in_dma(i):
            c, src, _, h0 = task(i)
            return pltpu.make_async_copy(
                in_hbm.at[c, src, pl.ds(h0, hh)], bufs.at[sid, i], sem_in.at[i]
            )

        def out_dma(i):
            c, _, j, h0 = task(i)
            return pltpu.make_async_copy(
                bufs.at[sid, i], out_hbm.at[c, j, pl.ds(h0, hh)], sem_out.at[i]
            )

        for i in range(tpw):
            in_dma(i).start()
        for i in range(tpw):
            in_dma(i).wait()
            out_dma(i).start()
        for i in range(tpw):
            out_dma(i).wait()

    return k(frames)


def kernel(frames):
    C, T, H, W = frames.shape
    n = T // _ALPHA
    idx = np.linspace(0.0, T - 1, n).astype(np.int32)
    assert all(int(t) == (j * (T - 1)) // (n - 1) for j, t in enumerate(idx))

    slow = _sc_gather(frames, C, T, n)
    fast = _tc_fast_copy(frames)
    return (slow, fast)


# speedup vs baseline: 1.0144x; 1.0028x over previous
import functools

import jax
import numpy as np
from jax import lax
from jax.experimental import pallas as pl
from jax.experimental.pallas import tpu as pltpu
from jax.experimental.pallas import tpu_sc as plsc

_ALPHA = 4
_NC, _NS = 2, 16
_NW = _NC * _NS


def _tc_fast_copy(frames):
    C, T, H, W = frames.shape
    n = T // _ALPHA

    def _body(in_hbm, fast_hbm, bufs, sem_in, sem_fast):
        def in_dma(j):
            return pltpu.make_async_copy(
                in_hbm.at[:, pl.ds(j * _ALPHA, _ALPHA)],
                bufs.at[j],
                sem_in.at[j],
            )

        def fast_dma(j):
            return pltpu.make_async_copy(
                bufs.at[j],
                fast_hbm.at[:, pl.ds(j * _ALPHA, _ALPHA)],
                sem_fast.at[j],
            )

        for j in range(n):
            in_dma(j).start()
        for j in range(n):
            in_dma(j).wait()
            fast_dma(j).start()
        for j in range(n):
            fast_dma(j).wait()

    return pl.pallas_call(
        _body,
        in_specs=[pl.BlockSpec(memory_space=pltpu.MemorySpace.HBM)],
        out_specs=pl.BlockSpec(memory_space=pltpu.MemorySpace.HBM),
        out_shape=jax.ShapeDtypeStruct((C, T, H, W), frames.dtype),
        scratch_shapes=[
            pltpu.VMEM((n, C, _ALPHA, H, W), frames.dtype),
            pltpu.SemaphoreType.DMA((n,)),
            pltpu.SemaphoreType.DMA((n,)),
        ],
    )(frames)


def _sc_gather(frames, C, T, n):
    _, _, H, W = frames.shape
    planes = C * n
    nchunk = 1
    while (planes * nchunk) % _NW or H % nchunk or (H // nchunk) % 8:
        nchunk += 1
    hh = H // nchunk
    tpw = planes * nchunk // _NW

    mesh = plsc.VectorSubcoreMesh(core_axis_name="c", subcore_axis_name="s")

    @functools.partial(
        pl.kernel,
        mesh=mesh,
        out_type=jax.ShapeDtypeStruct((C, n, H, W), frames.dtype),
        scratch_types=[
            pltpu.VMEM_SHARED((_NS, tpw, hh, W), frames.dtype),
            pltpu.SemaphoreType.DMA((tpw,)),
            pltpu.SemaphoreType.DMA((tpw,)),
        ],
        compiler_params=pltpu.CompilerParams(use_tc_tiling_on_sc=True),
    )
    def k(in_hbm, out_hbm, bufs, sem_in, sem_out):
        sid = lax.axis_index("s")
        wid = sid * _NC + lax.axis_index("c")

        def task(i):
            t = wid * tpw + i
            plane = t // nchunk
            chunk = t % nchunk
            j = plane % n
            c = plane // n
            src = (j * (T - 1)) // (n - 1)
            return c, src, j, chunk * hh

        def in_dma(i):
            c, src, _, h0 = task(i)
            return pltpu.make_async_copy(
                in_hbm.at[c, src, pl.ds(h0, hh)], bufs.at[sid, i], sem_in.at[i]
            )

        def out_dma(i):
            c, _, j, h0 = task(i)
            return pltpu.make_async_copy(
                bufs.at[sid, i], out_hbm.at[c, j, pl.ds(h0, hh)], sem_out.at[i]
            )

        for i in range(tpw):
            in_dma(i).start()
        for i in range(tpw):
            in_dma(i).wait()
            out_dma(i).start()
        for i in range(tpw):
            out_dma(i).wait()

    return k(frames)


def kernel(frames):
    C, T, H, W = frames.shape
    n = T // _ALPHA
    idx = np.linspace(0.0, T - 1, n).astype(np.int32)
    assert all(int(t) == (j * (T - 1)) // (n - 1) for j, t in enumerate(idx))

    slow = _sc_gather(frames, C, T, n)
    fast = _tc_fast_copy(frames)
    return (slow, fast)
